# baseline (speedup 1.0000x reference)
import functools
import jax
import jax.numpy as jnp
from jax import lax
from jax.experimental import pallas as pl
from jax.experimental.pallas import tpu as pltpu
from jax.experimental.pallas import tpu_sc as plsc

_G = 64
_NG = 26
_W = _G * _NG
_B = 16384
_PERM = [_NG - 1 - j for j in range(_NG)]

_B_SC = 6144
_B_TC = _B - _B_SC

_NC, _NS = 2, 16
_NW = _NC * _NS
_RPW = _B_SC // _NW
_CH = 32
_NCHUNK = _RPW // _CH
_L = 16

_mesh = plsc.VectorSubcoreMesh(core_axis_name="c", subcore_axis_name="s")


@functools.partial(
    pl.kernel,
    out_type=jax.ShapeDtypeStruct((_B_SC, _W), jnp.float32),
    mesh=_mesh,
    scratch_types=[
        pltpu.VMEM((2, _CH, _W), jnp.float32),
        pltpu.SemaphoreType.DMA,
        pltpu.SemaphoreType.DMA,
        pltpu.SemaphoreType.DMA,
        pltpu.SemaphoreType.DMA,
    ],
    compiler_params=pltpu.CompilerParams(use_tc_tiling_on_sc=True),
)
def _sc_permute(in_hbm, out_hbm, buf, sem_in0, sem_in1, sem_out0, sem_out1):
    wid = lax.axis_index("s") * _NC + lax.axis_index("c")
    row0 = _B_TC + wid * _RPW

    sem_in = (sem_in0, sem_in1)
    sem_out = (sem_out0, sem_out1)

    def in_copy(c, b):
        r = row0 + c * _CH
        return pltpu.make_async_copy(in_hbm.at[pl.ds(r, _CH)], buf.at[b], sem_in[b])

    def out_copy(c, b):
        r = wid * _RPW + c * _CH
        return pltpu.make_async_copy(buf.at[b], out_hbm.at[pl.ds(r, _CH)], sem_out[b])

    def permute(b):
        @pl.loop(0, _CH)
        def _row(r):
            for g in range(_NG // 2):
                o1 = _G * g
                o2 = _G * (_NG - 1 - g)
                for i in range(_G // _L):
                    s1 = pl.ds(o1 + _L * i, _L)
                    s2 = pl.ds(o2 + _L * i, _L)
                    a = buf[b, r, s1]
                    z = buf[b, r, s2]
                    buf[b, r, s2] = a
                    buf[b, r, s1] = z

    in_copy(0, 0).start()
    in_copy(1, 1).start()

    @pl.loop(0, _NCHUNK, step=2)
    def _pair(k):
        for b in range(2):
            c = k + b
            in_copy(c, b).wait()
            permute(b)
            out_copy(c, b).start()

        @pl.when(k + 2 < _NCHUNK)
        def _():
            for b in range(2):
                out_copy(k + b, b).wait()
                in_copy(k + 2 + b, b).start()

    out_copy(_NCHUNK - 2, 0).wait()
    out_copy(_NCHUNK - 1, 1).wait()


_TC_BLK = 512


def _tc_body(in_ref, out_ref):
    x = in_ref[...]
    parts = [x[:, _G * p:_G * (p + 1)] for p in _PERM]
    out_ref[...] = jnp.concatenate(parts, axis=1)


def _tc_permute(x_full):
    grid = (_B_TC // _TC_BLK,)
    return pl.pallas_call(
        _tc_body,
        grid=grid,
        in_specs=[pl.BlockSpec((_TC_BLK, _W), lambda i: (i, 0))],
        out_specs=pl.BlockSpec((_TC_BLK, _W), lambda i: (i, 0)),
        out_shape=jax.ShapeDtypeStruct((_B_TC, _W), x_full.dtype),
    )(x_full)


def kernel(pooled_embs):
    sc_out = _sc_permute(pooled_embs)
    tc_out = _tc_permute(pooled_embs)
    return jnp.concatenate([tc_out, sc_out], axis=0)


# speedup vs baseline: 1.5005x; 1.5005x over previous
import functools
import jax
import jax.numpy as jnp
from jax import lax
from jax.experimental import pallas as pl
from jax.experimental.pallas import tpu as pltpu
from jax.experimental.pallas import tpu_sc as plsc

_G = 64
_NG = 26
_W = _G * _NG
_B = 16384
_NC, _NS = 2, 16
_NW = _NC * _NS
_RPW = _B // _NW
_CH = 16
_NCHUNK = _RPW // _CH
_L = 16

_mesh = plsc.VectorSubcoreMesh(core_axis_name="c", subcore_axis_name="s")


@functools.partial(
    pl.kernel,
    out_type=jax.ShapeDtypeStruct((_B, _W), jnp.float32),
    mesh=_mesh,
    scratch_types=[
        pltpu.VMEM((4, _CH, _W), jnp.float32),
        pltpu.SemaphoreType.DMA,
        pltpu.SemaphoreType.DMA,
        pltpu.SemaphoreType.DMA,
        pltpu.SemaphoreType.DMA,
        pltpu.SemaphoreType.DMA,
        pltpu.SemaphoreType.DMA,
        pltpu.SemaphoreType.DMA,
        pltpu.SemaphoreType.DMA,
    ],
    compiler_params=pltpu.CompilerParams(use_tc_tiling_on_sc=True),
)
def _sc_permute(in_hbm, out_hbm, buf, si0, si1, si2, si3, so0, so1, so2, so3):
    wid = lax.axis_index("s") * _NC + lax.axis_index("c")
    row0 = wid * _RPW
    sem_in = (si0, si1, si2, si3)
    sem_out = (so0, so1, so2, so3)

    def in_copy(c, b):
        r = row0 + c * _CH
        return pltpu.make_async_copy(in_hbm.at[pl.ds(r, _CH)], buf.at[b], sem_in[b])

    def out_copy(c, b):
        r = row0 + c * _CH
        return pltpu.make_async_copy(buf.at[b], out_hbm.at[pl.ds(r, _CH)], sem_out[b])

    def permute(b):
        @pl.loop(0, _CH)
        def _row(r):
            for g in range(_NG // 2):
                o1 = _G * g
                o2 = _G * (_NG - 1 - g)
                for i in range(_G // _L):
                    s1 = pl.ds(o1 + _L * i, _L)
                    s2 = pl.ds(o2 + _L * i, _L)
                    a = buf[b, r, s1]
                    z = buf[b, r, s2]
                    buf[b, r, s2] = a
                    buf[b, r, s1] = z

    in_copy(0, 0).start()
    in_copy(1, 1).start()

    @pl.loop(0, _NCHUNK, step=4)
    def _quad(k):
        for b in range(4):
            c = k + b
            in_copy(c, b).wait()
            permute(b)
            out_copy(c, b).start()

            @pl.when(c >= 2)
            def _():
                out_copy(c - 2, (b - 2) % 4).wait()

            @pl.when(c + 2 < _NCHUNK)
            def _():
                in_copy(c + 2, (b + 2) % 4).start()

    out_copy(_NCHUNK - 2, (_NCHUNK - 2) % 4).wait()
    out_copy(_NCHUNK - 1, (_NCHUNK - 1) % 4).wait()


def kernel(pooled_embs):
    return _sc_permute(pooled_embs)


# speedup vs baseline: 1.5569x; 1.0376x over previous
import functools
import jax
import jax.numpy as jnp
from jax import lax
from jax.experimental import pallas as pl
from jax.experimental.pallas import tpu as pltpu
from jax.experimental.pallas import tpu_sc as plsc

_G = 64
_NG = 26
_W = _G * _NG
_B = 16384
_NC, _NS = 2, 16
_NW = _NC * _NS
_RPW = _B // _NW
_CH = 32
_NCHUNK = _RPW // _CH
_L = 16

_mesh = plsc.VectorSubcoreMesh(core_axis_name="c", subcore_axis_name="s")


@functools.partial(
    pl.kernel,
    out_type=jax.ShapeDtypeStruct((_B, _W), jnp.float32),
    mesh=_mesh,
    scratch_types=[
        pltpu.VMEM((2, _CH, _W), jnp.float32),
        pltpu.SemaphoreType.DMA,
        pltpu.SemaphoreType.DMA,
        pltpu.SemaphoreType.DMA,
        pltpu.SemaphoreType.DMA,
    ],
    compiler_params=pltpu.CompilerParams(use_tc_tiling_on_sc=False),
)
def _sc_permute(in_hbm, out_hbm, buf, sem_in0, sem_in1, sem_out0, sem_out1):
    wid = lax.axis_index("s") * _NC + lax.axis_index("c")
    row0 = wid * _RPW
    sem_in = (sem_in0, sem_in1)
    sem_out = (sem_out0, sem_out1)

    def in_copy(c, b):
        r = row0 + c * _CH
        return pltpu.make_async_copy(in_hbm.at[pl.ds(r, _CH)], buf.at[b], sem_in[b])

    def out_copy(c, b):
        r = row0 + c * _CH
        return pltpu.make_async_copy(buf.at[b], out_hbm.at[pl.ds(r, _CH)], sem_out[b])

    def permute(b):
        @pl.loop(0, _CH)
        def _row(r):
            for g in range(_NG // 2):
                o1 = _G * g
                o2 = _G * (_NG - 1 - g)
                for i in range(_G // _L):
                    s1 = pl.ds(o1 + _L * i, _L)
                    s2 = pl.ds(o2 + _L * i, _L)
                    a = buf[b, r, s1]
                    z = buf[b, r, s2]
                    buf[b, r, s2] = a
                    buf[b, r, s1] = z

    in_copy(0, 0).start()
    in_copy(1, 1).start()

    @pl.loop(0, _NCHUNK, step=2)
    def _pair(k):
        for b in range(2):
            c = k + b
            in_copy(c, b).wait()
            permute(b)
            out_copy(c, b).start()

        @pl.when(k + 2 < _NCHUNK)
        def _():
            for b in range(2):
                out_copy(k + b, b).wait()
                in_copy(k + 2 + b, b).start()

    out_copy(_NCHUNK - 2, 0).wait()
    out_copy(_NCHUNK - 1, 1).wait()


def kernel(pooled_embs):
    return _sc_permute(pooled_embs)
